# Initial kernel scaffold; baseline (speedup 1.0000x reference)
import jax
import jax.numpy as jnp
from jax.experimental import pallas as pl


def kernel(x, edge_index, W1, b1, W2, b2, W3, b3):
    raise NotImplementedError("write your pallas kernel here")


# speedup vs baseline: 4.9632x; 4.9632x over previous
import functools

import jax
import jax.numpy as jnp
from jax import lax
from jax.experimental import pallas as pl
from jax.experimental.pallas import tpu as pltpu
from jax.experimental.pallas import tpu_sc as plsc

N_NODES = 10000
N_PAD = 10240
N_EDGES = 160000
D = 128

NC = 2
NS = 16
NW = NC * NS
E_PER_TILE = N_EDGES // NW
CHUNK = 128
N_FULL = E_PER_TILE // CHUNK
TAIL = E_PER_TILE - N_FULL * CHUNK
ROWS_PER_TILE = N_PAD // NS


def _mesh():
    return plsc.VectorSubcoreMesh(core_axis_name="c", subcore_axis_name="s")


def _zero_fill(ref, rows, width):
    z = jnp.zeros((16,), jnp.float32)

    def body(i, _):
        for j in range(width // 16):
            ref[i, pl.ds(j * 16, 16)] = z
        return 0

    lax.fori_loop(0, rows, body, 0)


@functools.lru_cache(maxsize=None)
def _deg_kernel():
    @functools.partial(
        pl.kernel,
        out_type=jax.ShapeDtypeStruct((NC, N_PAD, D), jnp.float32),
        mesh=_mesh(),
        scratch_types=[
            pltpu.VMEM((CHUNK,), jnp.int32),
            pltpu.VMEM((TAIL,), jnp.int32),
            pltpu.VMEM((CHUNK, D), jnp.float32),
            pltpu.VMEM((TAIL, D), jnp.float32),
            pltpu.VMEM_SHARED((N_PAD, D), jnp.float32),
        ],
    )
    def deg(dst_hbm, out_hbm, idx_v, idx_t, ones_v, ones_t, acc):
        cid = lax.axis_index("c")
        sid = lax.axis_index("s")
        base = (cid * NS + sid) * E_PER_TILE

        _zero_fill(ones_v, CHUNK, D)
        r0 = sid * ROWS_PER_TILE
        for k in range(ROWS_PER_TILE // CHUNK):
            pltpu.sync_copy(ones_v, acc.at[pl.ds(r0 + k * CHUNK, CHUNK)])
        one = jnp.ones((16,), jnp.float32)

        def fill_one(i, _):
            for j in range(D // 16):
                ones_v[i, pl.ds(j * 16, 16)] = one
            return 0

        lax.fori_loop(0, CHUNK, fill_one, 0)
        for i in range(TAIL):
            for j in range(D // 16):
                ones_t[i, pl.ds(j * 16, 16)] = one
        plsc.subcore_barrier()

        def edge_chunk(j, _):
            pltpu.sync_copy(dst_hbm.at[pl.ds(base + j * CHUNK, CHUNK)], idx_v)
            pltpu.sync_copy(ones_v, acc.at[idx_v], add=True)
            return 0

        lax.fori_loop(0, N_FULL, edge_chunk, 0)
        pltpu.sync_copy(dst_hbm.at[pl.ds(base + N_FULL * CHUNK, TAIL)], idx_t)
        pltpu.sync_copy(ones_t, acc.at[idx_t], add=True)
        plsc.subcore_barrier()

        pltpu.sync_copy(acc.at[pl.ds(r0, ROWS_PER_TILE)],
                        out_hbm.at[cid, pl.ds(r0, ROWS_PER_TILE)])

    return deg


@functools.lru_cache(maxsize=None)
def _agg_kernel():
    @functools.partial(
        pl.kernel,
        out_type=jax.ShapeDtypeStruct((NC, N_PAD, D), jnp.float32),
        mesh=_mesh(),
        scratch_types=[
            pltpu.VMEM((CHUNK,), jnp.int32),
            pltpu.VMEM((CHUNK,), jnp.int32),
            pltpu.VMEM((TAIL,), jnp.int32),
            pltpu.VMEM((TAIL,), jnp.int32),
            pltpu.VMEM((CHUNK, D), jnp.float32),
            pltpu.VMEM((TAIL, D), jnp.float32),
            pltpu.VMEM_SHARED((N_PAD, D), jnp.float32),
            pltpu.SemaphoreType.DMA,
        ],
    )
    def agg(x2_hbm, src_hbm, dst_hbm, out_hbm,
            sidx_v, didx_v, sidx_t, didx_t, rows_v, rows_t, acc, sem):
        cid = lax.axis_index("c")
        sid = lax.axis_index("s")
        base = (cid * NS + sid) * E_PER_TILE

        _zero_fill(rows_v, CHUNK, D)
        r0 = sid * ROWS_PER_TILE
        for k in range(ROWS_PER_TILE // CHUNK):
            pltpu.sync_copy(rows_v, acc.at[pl.ds(r0 + k * CHUNK, CHUNK)])
        plsc.subcore_barrier()

        def edge_chunk(j, _):
            off = base + j * CHUNK
            pltpu.sync_copy(src_hbm.at[pl.ds(off, CHUNK)], sidx_v)
            pltpu.sync_copy(dst_hbm.at[pl.ds(off, CHUNK)], didx_v)
            pltpu.async_copy(x2_hbm.at[sidx_v], rows_v, sem).wait()
            pltpu.sync_copy(rows_v, acc.at[didx_v], add=True)
            return 0

        lax.fori_loop(0, N_FULL, edge_chunk, 0)
        off_t = base + N_FULL * CHUNK
        pltpu.sync_copy(src_hbm.at[pl.ds(off_t, TAIL)], sidx_t)
        pltpu.sync_copy(dst_hbm.at[pl.ds(off_t, TAIL)], didx_t)
        pltpu.async_copy(x2_hbm.at[sidx_t], rows_t, sem).wait()
        pltpu.sync_copy(rows_t, acc.at[didx_t], add=True)
        plsc.subcore_barrier()

        pltpu.sync_copy(acc.at[pl.ds(r0, ROWS_PER_TILE)],
                        out_hbm.at[cid, pl.ds(r0, ROWS_PER_TILE)])

    return agg


BM = 1000


def _tc_body(ap_ref, dp_ref, x1_ref, w_ref, b_ref, out_ref):
    a = ap_ref[0] + ap_ref[1]
    d = dp_ref[0] + dp_ref[1]
    inv = 1.0 / jnp.maximum(d, 1.0)
    h = jnp.dot(a * inv, w_ref[...], preferred_element_type=jnp.float32)
    out_ref[...] = jnp.maximum(h + b_ref[...], 0.0) + x1_ref[...]


@functools.lru_cache(maxsize=None)
def _tc_kernel():
    grid = (N_NODES // BM,)
    return pl.pallas_call(
        _tc_body,
        grid=grid,
        in_specs=[
            pl.BlockSpec((NC, BM, D), lambda i: (0, i, 0)),
            pl.BlockSpec((NC, BM, 1), lambda i: (0, i, 0)),
            pl.BlockSpec((BM, D), lambda i: (i, 0)),
            pl.BlockSpec((D, D), lambda i: (0, 0)),
            pl.BlockSpec((1, D), lambda i: (0, 0)),
        ],
        out_specs=pl.BlockSpec((BM, D), lambda i: (i, 0)),
        out_shape=jax.ShapeDtypeStruct((N_NODES, D), jnp.float32),
    )


def kernel(x, edge_index, W1, b1, W2, b2, W3, b3):
    src = edge_index[0]
    dst = edge_index[1]
    deg_col = _deg_kernel()(dst)[:, :, :1]
    x1 = x[:, :D]
    x2 = x[:, D:]
    tc = _tc_kernel()
    for W, b in ((W1, b1), (W2, b2), (W3, b3)):
        agg_parts = _agg_kernel()(x2, src, dst)
        y2 = tc(agg_parts, deg_col, x1, W, b.reshape(1, D))
        x1, x2 = x2, y2
    out = jnp.concatenate([x1, x2], axis=1)
    return (out, out)


# speedup vs baseline: 8.4755x; 1.7077x over previous
import functools

import jax
import jax.numpy as jnp
from jax import lax
from jax.experimental import pallas as pl
from jax.experimental.pallas import tpu as pltpu
from jax.experimental.pallas import tpu_sc as plsc

N_NODES = 10000
N_PAD = 10240
N_EDGES = 160000
D = 128

NC = 2
NS = 16
NW = NC * NS
E_PER_TILE = N_EDGES // NW
CHUNK = 128
N_FULL = E_PER_TILE // CHUNK
TAIL = E_PER_TILE - N_FULL * CHUNK
ROWS_PER_TILE = N_PAD // NS


def _mesh():
    return plsc.VectorSubcoreMesh(core_axis_name="c", subcore_axis_name="s")


def _zero_fill(ref, rows, width):
    z = jnp.zeros((16,), jnp.float32)

    def body(i, _):
        for j in range(width // 16):
            ref[i, pl.ds(j * 16, 16)] = z
        return 0

    lax.fori_loop(0, rows, body, 0)


def _ones_fill(ref, rows, width):
    one = jnp.ones((16,), jnp.float32)

    def body(i, _):
        for j in range(width // 16):
            ref[i, pl.ds(j * 16, 16)] = one
        return 0

    lax.fori_loop(0, rows, body, 0)


@functools.lru_cache(maxsize=None)
def _deg_kernel():
    @functools.partial(
        pl.kernel,
        out_type=jax.ShapeDtypeStruct((NC, N_PAD, D), jnp.float32),
        mesh=_mesh(),
        scratch_types=[
            pltpu.VMEM((N_FULL, CHUNK), jnp.int32),
            pltpu.VMEM((TAIL,), jnp.int32),
            pltpu.VMEM((CHUNK, D), jnp.float32),
            pltpu.VMEM((TAIL, D), jnp.float32),
            pltpu.VMEM_SHARED((N_PAD, D), jnp.float32),
            pltpu.SemaphoreType.DMA,
            pltpu.SemaphoreType.DMA,
        ],
    )
    def deg(dst_hbm, out_hbm, didx2, idx_t, ones_v, ones_t, acc, sem_i, sem_s):
        cid = lax.axis_index("c")
        sid = lax.axis_index("s")
        base = (cid * NS + sid) * E_PER_TILE

        idx_cps = [
            pltpu.async_copy(dst_hbm.at[pl.ds(base + j * CHUNK, CHUNK)],
                             didx2.at[j], sem_i)
            for j in range(N_FULL)
        ]
        idx_cps.append(
            pltpu.async_copy(dst_hbm.at[pl.ds(base + N_FULL * CHUNK, TAIL)],
                             idx_t, sem_i))

        _zero_fill(ones_v, CHUNK, D)
        r0 = sid * ROWS_PER_TILE
        for k in range(ROWS_PER_TILE // CHUNK):
            pltpu.sync_copy(ones_v, acc.at[pl.ds(r0 + k * CHUNK, CHUNK)])
        _ones_fill(ones_v, CHUNK, D)
        _ones_fill(ones_t, TAIL, D)
        for cp in idx_cps:
            cp.wait()
        plsc.subcore_barrier()

        sc_cps = [
            pltpu.async_copy(ones_v, acc.at[didx2.at[j]], sem_s, add=True)
            for j in range(N_FULL)
        ]
        sc_cps.append(pltpu.async_copy(ones_t, acc.at[idx_t], sem_s, add=True))
        for cp in sc_cps:
            cp.wait()
        plsc.subcore_barrier()

        pltpu.sync_copy(acc.at[pl.ds(r0, ROWS_PER_TILE)],
                        out_hbm.at[cid, pl.ds(r0, ROWS_PER_TILE)])

    return deg


@functools.lru_cache(maxsize=None)
def _agg_kernel():
    @functools.partial(
        pl.kernel,
        out_type=jax.ShapeDtypeStruct((NC, N_PAD, D), jnp.float32),
        mesh=_mesh(),
        scratch_types=[
            pltpu.VMEM((N_FULL * CHUNK,), jnp.int32),
            pltpu.VMEM((N_FULL, CHUNK), jnp.int32),
            pltpu.VMEM((TAIL,), jnp.int32),
            pltpu.VMEM((TAIL,), jnp.int32),
            pltpu.VMEM((CHUNK, D), jnp.float32),
            pltpu.VMEM((CHUNK, D), jnp.float32),
            pltpu.VMEM((TAIL, D), jnp.float32),
            pltpu.VMEM_SHARED((N_PAD, D), jnp.float32),
            pltpu.SemaphoreType.DMA,
            pltpu.SemaphoreType.DMA,
            pltpu.SemaphoreType.DMA,
        ],
    )
    def agg(x2_hbm, src_hbm, dst_hbm, out_hbm,
            sidx, didx2, sidx_t, didx_t, rows_a, rows_b, rows_t,
            acc, sem_a, sem_b, sem_i):
        cid = lax.axis_index("c")
        sid = lax.axis_index("s")
        base = (cid * NS + sid) * E_PER_TILE

        idx_cps = [
            pltpu.async_copy(src_hbm.at[pl.ds(base, N_FULL * CHUNK)], sidx,
                             sem_i),
            pltpu.async_copy(src_hbm.at[pl.ds(base + N_FULL * CHUNK, TAIL)],
                             sidx_t, sem_i),
            pltpu.async_copy(dst_hbm.at[pl.ds(base + N_FULL * CHUNK, TAIL)],
                             didx_t, sem_i),
        ]
        idx_cps += [
            pltpu.async_copy(dst_hbm.at[pl.ds(base + j * CHUNK, CHUNK)],
                             didx2.at[j], sem_i)
            for j in range(N_FULL)
        ]

        _zero_fill(rows_a, CHUNK, D)
        r0 = sid * ROWS_PER_TILE
        for k in range(ROWS_PER_TILE // CHUNK):
            pltpu.sync_copy(rows_a, acc.at[pl.ds(r0 + k * CHUNK, CHUNK)])
        for cp in idx_cps:
            cp.wait()
        plsc.subcore_barrier()

        def gather(j, buf, sem):
            return pltpu.async_copy(
                x2_hbm.at[sidx.at[pl.ds(j * CHUNK, CHUNK)]], buf, sem)

        gather(0, rows_a, sem_a).wait()

        def body(k, _):
            j0 = 2 * k
            j1 = j0 + 1
            gb = gather(j1, rows_b, sem_b)
            pltpu.sync_copy(rows_a, acc.at[didx2.at[j0]], add=True)
            ga = gather(j0 + 2, rows_a, sem_a)
            gb.wait()
            pltpu.sync_copy(rows_b, acc.at[didx2.at[j1]], add=True)
            ga.wait()
            return 0

        lax.fori_loop(0, (N_FULL - 1) // 2, body, 0)
        pltpu.async_copy(x2_hbm.at[sidx_t], rows_t, sem_b).wait()
        pltpu.sync_copy(rows_a, acc.at[didx2.at[N_FULL - 1]], add=True)
        pltpu.sync_copy(rows_t, acc.at[didx_t], add=True)
        plsc.subcore_barrier()

        pltpu.sync_copy(acc.at[pl.ds(r0, ROWS_PER_TILE)],
                        out_hbm.at[cid, pl.ds(r0, ROWS_PER_TILE)])

    return agg


BM = 1000


def _tc_body(ap_ref, dp_ref, x1_ref, w_ref, b_ref, out_ref):
    a = ap_ref[0] + ap_ref[1]
    d = dp_ref[0] + dp_ref[1]
    inv = 1.0 / jnp.maximum(d, 1.0)
    h = jnp.dot(a * inv, w_ref[...], preferred_element_type=jnp.float32)
    out_ref[...] = jnp.maximum(h + b_ref[...], 0.0) + x1_ref[...]


@functools.lru_cache(maxsize=None)
def _tc_kernel():
    grid = (N_NODES // BM,)
    return pl.pallas_call(
        _tc_body,
        grid=grid,
        in_specs=[
            pl.BlockSpec((NC, BM, D), lambda i: (0, i, 0)),
            pl.BlockSpec((NC, BM, 1), lambda i: (0, i, 0)),
            pl.BlockSpec((BM, D), lambda i: (i, 0)),
            pl.BlockSpec((D, D), lambda i: (0, 0)),
            pl.BlockSpec((1, D), lambda i: (0, 0)),
        ],
        out_specs=pl.BlockSpec((BM, D), lambda i: (i, 0)),
        out_shape=jax.ShapeDtypeStruct((N_NODES, D), jnp.float32),
    )


def kernel(x, edge_index, W1, b1, W2, b2, W3, b3):
    src = edge_index[0]
    dst = edge_index[1]
    deg_col = _deg_kernel()(dst)[:, :, :1]
    x1 = x[:, :D]
    x2 = x[:, D:]
    tc = _tc_kernel()
    for W, b in ((W1, b1), (W2, b2), (W3, b3)):
        agg_parts = _agg_kernel()(x2, src, dst)
        y2 = tc(agg_parts, deg_col, x1, W, b.reshape(1, D))
        x1, x2 = x2, y2
    out = jnp.concatenate([x1, x2], axis=1)
    return (out, out)
